# baseline (speedup 1.0000x reference)
import functools

import jax
import jax.numpy as jnp
from jax import lax
from jax.experimental import pallas as pl
from jax.experimental.pallas import tpu as pltpu
from jax.experimental.pallas import tpu_sc as plsc

_EPS = 1e-16


def _proj_body(x_ref, wa_ref, wb_ref, oa_ref, ob_ref):
    x = x_ref[...]
    oa_ref[...] = jnp.dot(x, wa_ref[...], preferred_element_type=jnp.float32)
    ob_ref[...] = jnp.dot(x, wb_ref[...], preferred_element_type=jnp.float32)


def _proj(x, wa, wb, blk=2000):
    n, k = x.shape
    da, db = wa.shape[1], wb.shape[1]
    return pl.pallas_call(
        _proj_body,
        grid=(n // blk,),
        in_specs=[
            pl.BlockSpec((blk, k), lambda i: (i, 0)),
            pl.BlockSpec((k, da), lambda i: (0, 0)),
            pl.BlockSpec((k, db), lambda i: (0, 0)),
        ],
        out_specs=[
            pl.BlockSpec((blk, da), lambda i: (i, 0)),
            pl.BlockSpec((blk, db), lambda i: (i, 0)),
        ],
        out_shape=[
            jax.ShapeDtypeStruct((n, da), jnp.float32),
            jax.ShapeDtypeStruct((n, db), jnp.float32),
        ],
    )(x, wa, wb)


def _norm_proj_body(feat_w, a0_ref, a1_ref, s_ref, b_ref, wa_ref, wb_ref,
                    oa_ref, ob_ref):
    hs = a0_ref[...] + a1_ref[...]
    den = jnp.dot(hs, s_ref[...], preferred_element_type=jnp.float32)
    h = jnp.maximum(hs[:, :feat_w] / (den + _EPS) + b_ref[...], 0.0)
    oa_ref[...] = jnp.dot(h, wa_ref[...], preferred_element_type=jnp.float32)
    ob_ref[...] = jnp.dot(h, wb_ref[...], preferred_element_type=jnp.float32)


def _norm_proj(a0, a1, sel, b, wa, wb, blk=2000):
    n, dt = a0.shape
    feat_w = sel.shape[1]
    da, db = wa.shape[1], wb.shape[1]
    return pl.pallas_call(
        functools.partial(_norm_proj_body, feat_w),
        grid=(n // blk,),
        in_specs=[
            pl.BlockSpec((blk, dt), lambda i: (i, 0)),
            pl.BlockSpec((blk, dt), lambda i: (i, 0)),
            pl.BlockSpec((dt, feat_w), lambda i: (0, 0)),
            pl.BlockSpec((1, feat_w), lambda i: (0, 0)),
            pl.BlockSpec((feat_w, da), lambda i: (0, 0)),
            pl.BlockSpec((feat_w, db), lambda i: (0, 0)),
        ],
        out_specs=[
            pl.BlockSpec((blk, da), lambda i: (i, 0)),
            pl.BlockSpec((blk, db), lambda i: (i, 0)),
        ],
        out_shape=[
            jax.ShapeDtypeStruct((n, da), jnp.float32),
            jax.ShapeDtypeStruct((n, db), jnp.float32),
        ],
    )(a0, a1, sel, b, wa, wb)


def _final_body(feat_w, a0_ref, a1_ref, s_ref, b_ref, o_ref):
    hs = a0_ref[...] + a1_ref[...]
    den = jnp.dot(hs, s_ref[...], preferred_element_type=jnp.float32)
    o_ref[...] = hs[:, :feat_w] / (den + _EPS) + b_ref[...]


def _final(a0, a1, sel, b, blk=2000):
    n, dt = a0.shape
    feat_w = sel.shape[1]
    return pl.pallas_call(
        functools.partial(_final_body, feat_w),
        grid=(n // blk,),
        in_specs=[
            pl.BlockSpec((blk, dt), lambda i: (i, 0)),
            pl.BlockSpec((blk, dt), lambda i: (i, 0)),
            pl.BlockSpec((dt, feat_w), lambda i: (0, 0)),
            pl.BlockSpec((1, feat_w), lambda i: (0, 0)),
        ],
        out_specs=pl.BlockSpec((blk, feat_w), lambda i: (i, 0)),
        out_shape=jax.ShapeDtypeStruct((n, feat_w), jnp.float32),
    )(a0, a1, sel, b)


def _make_sc_gat(n_nodes, dt, feat_cols, head_w, n_chunks, chunk):
    n_heads = feat_cols // head_w
    vregs_per_head = head_w // 16
    mesh = plsc.VectorSubcoreMesh(core_axis_name="c", subcore_axis_name="s")
    stripe = (n_nodes // 16) & ~7
    last_stripe = n_nodes - 15 * stripe

    @functools.partial(
        pl.kernel,
        mesh=mesh,
        compiler_params=pltpu.CompilerParams(use_tc_tiling_on_sc=False),
        out_type=jax.ShapeDtypeStruct((2, n_nodes, dt), jnp.float32),
        scratch_types=[
            pltpu.VMEM((3, chunk), jnp.int32),
            pltpu.VMEM((3, chunk), jnp.int32),
            pltpu.VMEM((3, chunk, dt), jnp.float32),
            pltpu.VMEM((3, chunk, 16), jnp.float32),
            pltpu.VMEM((3, chunk, dt), jnp.float32),
            pltpu.VMEM_SHARED((n_nodes, dt), jnp.float32),
            pltpu.SemaphoreType.DMA,
            pltpu.SemaphoreType.DMA,
            pltpu.SemaphoreType.DMA,
            pltpu.SemaphoreType.DMA,
            pltpu.SemaphoreType.DMA,
            pltpu.SemaphoreType.DMA,
            pltpu.SemaphoreType.DMA,
            pltpu.SemaphoreType.DMA,
            pltpu.SemaphoreType.DMA,
            pltpu.SemaphoreType.DMA,
            pltpu.SemaphoreType.DMA,
            pltpu.SemaphoreType.DMA,
        ],
    )
    def sc_gat(t_hbm, d_hbm, src_hbm, dst_hbm, out_hbm,
               sidx, didx, trows, drows, srows, acc,
               gt0, gt1, gt2, gd0, gd1, gd2,
               ss0, ss1, ss2, is0, is1, is2):
        c = lax.axis_index("c")
        s = lax.axis_index("s")
        wid = s * 2 + c
        gts = (gt0, gt1, gt2)
        gds = (gd0, gd1, gd2)
        sss = (ss0, ss1, ss2)
        iss = (is0, is1, is2)

        def idx_stage(ch, b):
            pltpu.async_copy(src_hbm.at[wid, ch], sidx.at[b], iss[b])
            pltpu.async_copy(dst_hbm.at[wid, ch], didx.at[b], iss[b])

        def idx_wait(b):
            pltpu.make_async_copy(src_hbm.at[wid, 0], sidx.at[b],
                                  iss[b]).wait()
            pltpu.make_async_copy(dst_hbm.at[wid, 0], didx.at[b],
                                  iss[b]).wait()

        def gather_start(ch, b):
            pltpu.async_copy(t_hbm.at[sidx.at[b]], trows.at[b], gts[b])
            pltpu.async_copy(d_hbm.at[didx.at[b]], drows.at[b], gds[b])

        def gather_wait(b):
            pltpu.make_async_copy(t_hbm.at[sidx.at[b]], trows.at[b],
                                  gts[b]).wait()
            pltpu.make_async_copy(d_hbm.at[didx.at[b]], drows.at[b],
                                  gds[b]).wait()

        def scatter_start(b):
            pltpu.async_copy(srows.at[b], acc.at[didx.at[b]], sss[b],
                             add=True)

        def scatter_wait(b):
            pltpu.make_async_copy(srows.at[b], acc.at[didx.at[b]],
                                  sss[b]).wait()

        idx_stage(0, 0)
        idx_stage(1, 1)

        zero16 = jnp.zeros((16,), jnp.float32)

        def zrow(r, _):
            for k16 in range(dt // 16):
                srows[0, r, pl.ds(k16 * 16, 16)] = zero16
            return 0

        lax.fori_loop(0, 16, zrow, 0)
        base = s * stripe
        n16 = jnp.where(s == 15, last_stripe // 16, stripe // 16)

        def zcp(j, _):
            pltpu.sync_copy(srows.at[0, pl.ds(0, 16)],
                            acc.at[pl.ds(base + j * 16, 16)])
            return 0

        lax.fori_loop(0, n16, zcp, 0)
        plsc.subcore_barrier()

        def compute(b):
            @plsc.parallel_loop(0, chunk, unroll=8)
            def edge_body(i):
                sv = trows[b, i, pl.ds(feat_cols, 16)]
                dv = drows[b, i, pl.ds(0, 16)]
                t = sv + dv
                w = jnp.exp(jnp.where(t >= 0, t, 0.2 * t))
                srows[b, i, pl.ds(feat_cols, 16)] = w
                for h in range(n_heads):
                    bj = w[h]
                    for v in range(vregs_per_head):
                        col = (h * vregs_per_head + v) * 16
                        srows[b, i, pl.ds(col, 16)] = (
                            trows[b, i, pl.ds(col, 16)] * bj)

        idx_wait(0)
        gather_start(0, 0)

        def sub_iter(ch, b):
            bn = (b + 1) % 3
            bp = (b + 2) % 3
            gather_wait(b)

            @pl.when(ch + 1 < n_chunks)
            def _():
                idx_wait(bn)
                gather_start(ch + 1, bn)

            compute(b)

            @pl.when(ch > 0)
            def _():
                scatter_wait(bp)

            scatter_start(b)

            @pl.when(ch + 2 < n_chunks)
            def _():
                idx_stage(ch + 2, bp)

        def triple_body(t3, _):
            for b in (0, 1, 2):
                sub_iter(3 * t3 + b, b)
            return 0

        n_triples = n_chunks // 3
        lax.fori_loop(0, n_triples, triple_body, 0)
        for ch in range(3 * n_triples, n_chunks):
            b = ch % 3
            bn = (b + 1) % 3
            bp = (b + 2) % 3
            gather_wait(b)
            if ch + 1 < n_chunks:
                idx_wait(bn)
                gather_start(ch + 1, bn)
            compute(b)
            if ch > 0:
                scatter_wait(bp)
            scatter_start(b)
            if ch + 2 < n_chunks:
                idx_stage(ch + 2, bp)
        scatter_wait((n_chunks - 1) % 3)
        plsc.subcore_barrier()

        def ocp(j, _):
            pltpu.sync_copy(acc.at[pl.ds(base + j * 16, 16)],
                            out_hbm.at[c, pl.ds(base + j * 16, 16)])
            return 0

        lax.fori_loop(0, n16, ocp, 0)

    return sc_gat


def _head_matrix(a):
    heads, ch = a.shape
    return (jnp.eye(heads, dtype=a.dtype)[:, None, :]
            * a[:, :, None]).reshape(heads * ch, heads)


def kernel(x, edge_index, W1, a_src1, a_dst1, b1, W2, a_src2, a_dst2, b2):
    n = x.shape[0]
    e = edge_index.shape[1]
    heads, hid = a_src1.shape
    feat1 = heads * hid
    out_dim = W2.shape[1]
    dt1, dt2 = feat1 + 16, out_dim + 16

    workers = 32
    chunk = 40
    n_chunks = e // (workers * chunk)

    src_w = edge_index[0].reshape(workers, n_chunks, chunk)
    dst_w = edge_index[1].reshape(workers, n_chunks, chunk)

    f32 = jnp.float32
    z8 = jnp.zeros((x.shape[1], 16 - heads), f32)
    wa1 = jnp.concatenate([W1, W1 @ _head_matrix(a_src1), z8], axis=1)
    wd1 = jnp.concatenate([W1 @ _head_matrix(a_dst1), z8], axis=1)

    t1, d1 = _proj(x, wa1, wd1)
    acc1 = _make_sc_gat(n, dt1, feat1, hid, n_chunks, chunk)(
        t1, d1, src_w, dst_w)

    sel1 = jnp.concatenate(
        [jnp.zeros((feat1, feat1), f32),
         jnp.repeat(jnp.eye(heads, dtype=f32), hid, axis=1),
         jnp.zeros((16 - heads, feat1), f32)], axis=0)

    z15 = jnp.zeros((feat1, 15), f32)
    wa2 = jnp.concatenate([W2, W2 @ a_src2.T, z15], axis=1)
    wd2 = jnp.concatenate([W2 @ a_dst2.T, z15], axis=1)

    t2, d2 = _norm_proj(acc1[0], acc1[1], sel1, b1.reshape(1, feat1),
                        wa2, wd2)
    acc2 = _make_sc_gat(n, dt2, out_dim, out_dim, n_chunks, chunk)(
        t2, d2, src_w, dst_w)

    sel2 = jnp.concatenate(
        [jnp.zeros((out_dim, out_dim), f32),
         jnp.ones((1, out_dim), f32),
         jnp.zeros((15, out_dim), f32)], axis=0)

    return _final(acc2[0], acc2[1], sel2, b2.reshape(1, out_dim))


# speedup vs baseline: 1.1888x; 1.1888x over previous
import functools

import jax
import jax.numpy as jnp
from jax import lax
from jax.experimental import pallas as pl
from jax.experimental.pallas import tpu as pltpu
from jax.experimental.pallas import tpu_sc as plsc

_EPS = 1e-16


def _proj_body(x_ref, wa_ref, wb_ref, oa_ref, ob_ref):
    x = x_ref[...]
    oa_ref[...] = jnp.dot(x, wa_ref[...], preferred_element_type=jnp.float32)
    ob_ref[...] = jnp.dot(x, wb_ref[...], preferred_element_type=jnp.float32)


def _proj(x, wa, wb, blk=2000):
    n, k = x.shape
    da, db = wa.shape[1], wb.shape[1]
    return pl.pallas_call(
        _proj_body,
        grid=(n // blk,),
        in_specs=[
            pl.BlockSpec((blk, k), lambda i: (i, 0)),
            pl.BlockSpec((k, da), lambda i: (0, 0)),
            pl.BlockSpec((k, db), lambda i: (0, 0)),
        ],
        out_specs=[
            pl.BlockSpec((blk, da), lambda i: (i, 0)),
            pl.BlockSpec((blk, db), lambda i: (i, 0)),
        ],
        out_shape=[
            jax.ShapeDtypeStruct((n, da), jnp.float32),
            jax.ShapeDtypeStruct((n, db), jnp.float32),
        ],
    )(x, wa, wb)


def _norm_proj_body(feat_w, a0_ref, a1_ref, s_ref, b_ref, wa_ref, wb_ref,
                    oa_ref, ob_ref):
    hs = a0_ref[...] + a1_ref[...]
    den = jnp.dot(hs, s_ref[...], preferred_element_type=jnp.float32)
    h = jnp.maximum(hs[:, :feat_w] / (den + _EPS) + b_ref[...], 0.0)
    oa_ref[...] = jnp.dot(h, wa_ref[...], preferred_element_type=jnp.float32)
    ob_ref[...] = jnp.dot(h, wb_ref[...], preferred_element_type=jnp.float32)


def _norm_proj(a0, a1, sel, b, wa, wb, blk=2000):
    n, dt = a0.shape
    feat_w = sel.shape[1]
    da, db = wa.shape[1], wb.shape[1]
    return pl.pallas_call(
        functools.partial(_norm_proj_body, feat_w),
        grid=(n // blk,),
        in_specs=[
            pl.BlockSpec((blk, dt), lambda i: (i, 0)),
            pl.BlockSpec((blk, dt), lambda i: (i, 0)),
            pl.BlockSpec((dt, feat_w), lambda i: (0, 0)),
            pl.BlockSpec((1, feat_w), lambda i: (0, 0)),
            pl.BlockSpec((feat_w, da), lambda i: (0, 0)),
            pl.BlockSpec((feat_w, db), lambda i: (0, 0)),
        ],
        out_specs=[
            pl.BlockSpec((blk, da), lambda i: (i, 0)),
            pl.BlockSpec((blk, db), lambda i: (i, 0)),
        ],
        out_shape=[
            jax.ShapeDtypeStruct((n, da), jnp.float32),
            jax.ShapeDtypeStruct((n, db), jnp.float32),
        ],
    )(a0, a1, sel, b, wa, wb)


def _final_body(feat_w, a0_ref, a1_ref, s_ref, b_ref, o_ref):
    hs = a0_ref[...] + a1_ref[...]
    den = jnp.dot(hs, s_ref[...], preferred_element_type=jnp.float32)
    o_ref[...] = hs[:, :feat_w] / (den + _EPS) + b_ref[...]


def _final(a0, a1, sel, b, blk=2000):
    n, dt = a0.shape
    feat_w = sel.shape[1]
    return pl.pallas_call(
        functools.partial(_final_body, feat_w),
        grid=(n // blk,),
        in_specs=[
            pl.BlockSpec((blk, dt), lambda i: (i, 0)),
            pl.BlockSpec((blk, dt), lambda i: (i, 0)),
            pl.BlockSpec((dt, feat_w), lambda i: (0, 0)),
            pl.BlockSpec((1, feat_w), lambda i: (0, 0)),
        ],
        out_specs=pl.BlockSpec((blk, feat_w), lambda i: (i, 0)),
        out_shape=jax.ShapeDtypeStruct((n, feat_w), jnp.float32),
    )(a0, a1, sel, b)


def _make_sc_gat(n_nodes, dt, feat_cols, head_w, n_chunks, chunk):
    n_heads = feat_cols // head_w
    vregs_per_head = head_w // 16
    mesh = plsc.VectorSubcoreMesh(core_axis_name="c", subcore_axis_name="s")
    stripe = (n_nodes // 16) & ~7
    last_stripe = n_nodes - 15 * stripe

    @functools.partial(
        pl.kernel,
        mesh=mesh,
        compiler_params=pltpu.CompilerParams(use_tc_tiling_on_sc=False),
        out_type=jax.ShapeDtypeStruct((2, n_nodes, dt), jnp.float32),
        scratch_types=[
            pltpu.VMEM((3, chunk), jnp.int32),
            pltpu.VMEM((3, chunk), jnp.int32),
            pltpu.VMEM((3, chunk, dt), jnp.float32),
            pltpu.VMEM((3, chunk, 16), jnp.float32),
            pltpu.VMEM((3, chunk, dt), jnp.float32),
            pltpu.VMEM_SHARED((n_nodes, dt), jnp.float32),
            pltpu.SemaphoreType.DMA,
            pltpu.SemaphoreType.DMA,
            pltpu.SemaphoreType.DMA,
            pltpu.SemaphoreType.DMA,
            pltpu.SemaphoreType.DMA,
            pltpu.SemaphoreType.DMA,
            pltpu.SemaphoreType.DMA,
            pltpu.SemaphoreType.DMA,
            pltpu.SemaphoreType.DMA,
            pltpu.SemaphoreType.DMA,
            pltpu.SemaphoreType.DMA,
            pltpu.SemaphoreType.DMA,
        ],
    )
    def sc_gat(t_hbm, d_hbm, src_hbm, dst_hbm, zeros_hbm, out_hbm,
               sidx, didx, trows, drows, srows, acc,
               gt0, gt1, gt2, gd0, gd1, gd2,
               ss0, ss1, ss2, is0, is1, is2):
        c = lax.axis_index("c")
        s = lax.axis_index("s")
        wid = s * 2 + c
        gts = (gt0, gt1, gt2)
        gds = (gd0, gd1, gd2)
        sss = (ss0, ss1, ss2)
        iss = (is0, is1, is2)

        def idx_stage(ch, b):
            pltpu.async_copy(src_hbm.at[wid, ch], sidx.at[b], iss[b])
            pltpu.async_copy(dst_hbm.at[wid, ch], didx.at[b], iss[b])

        def idx_wait(b):
            pltpu.make_async_copy(src_hbm.at[wid, 0], sidx.at[b],
                                  iss[b]).wait()
            pltpu.make_async_copy(dst_hbm.at[wid, 0], didx.at[b],
                                  iss[b]).wait()

        def gather_start(ch, b):
            pltpu.async_copy(t_hbm.at[sidx.at[b]], trows.at[b], gts[b])
            pltpu.async_copy(d_hbm.at[didx.at[b]], drows.at[b], gds[b])

        def gather_wait(b):
            pltpu.make_async_copy(t_hbm.at[sidx.at[b]], trows.at[b],
                                  gts[b]).wait()
            pltpu.make_async_copy(d_hbm.at[didx.at[b]], drows.at[b],
                                  gds[b]).wait()

        def scatter_start(b):
            pltpu.async_copy(srows.at[b], acc.at[didx.at[b]], sss[b],
                             add=True)

        def scatter_wait(b):
            pltpu.make_async_copy(srows.at[b], acc.at[didx.at[b]],
                                  sss[b]).wait()

        idx_stage(0, 0)
        idx_stage(1, 1)

        base = s * stripe

        @pl.when(s < 15)
        def _():
            pltpu.sync_copy(zeros_hbm.at[pl.ds(0, stripe)],
                            acc.at[pl.ds(base, stripe)])

        @pl.when(s == 15)
        def _():
            pltpu.sync_copy(zeros_hbm, acc.at[pl.ds(base, last_stripe)])

        plsc.subcore_barrier()

        def compute(b):
            @plsc.parallel_loop(0, chunk, unroll=8)
            def edge_body(i):
                sv = trows[b, i, pl.ds(feat_cols, 16)]
                dv = drows[b, i, pl.ds(0, 16)]
                t = sv + dv
                w = jnp.exp(jnp.where(t >= 0, t, 0.2 * t))
                srows[b, i, pl.ds(feat_cols, 16)] = w
                for h in range(n_heads):
                    bj = w[h]
                    for v in range(vregs_per_head):
                        col = (h * vregs_per_head + v) * 16
                        srows[b, i, pl.ds(col, 16)] = (
                            trows[b, i, pl.ds(col, 16)] * bj)

        idx_wait(0)
        gather_start(0, 0)

        def sub_iter(ch, b, is_static):
            bn = (b + 1) % 3
            bp = (b + 2) % 3
            gather_wait(b)
            if not is_static or ch + 1 < n_chunks:
                idx_wait(bn)
                gather_start(ch + 1, bn)
            compute(b)
            if not is_static or ch > 0:
                scatter_wait(bp)
            scatter_start(b)
            if not is_static or ch + 2 < n_chunks:
                idx_stage(ch + 2, bp)

        for ch in range(0, 3):
            sub_iter(ch, ch % 3, True)

        def triple_body(t3, _):
            for b in (0, 1, 2):
                sub_iter(3 * t3 + b, b, False)
            return 0

        epi = 3 * (n_chunks // 3 - 1)
        lax.fori_loop(1, n_chunks // 3 - 1, triple_body, 0)
        for ch in range(epi, n_chunks):
            sub_iter(ch, ch % 3, True)
        scatter_wait((n_chunks - 1) % 3)
        plsc.subcore_barrier()

        @pl.when(s < 15)
        def _():
            pltpu.sync_copy(acc.at[pl.ds(base, stripe)],
                            out_hbm.at[c, pl.ds(base, stripe)])

        @pl.when(s == 15)
        def _():
            pltpu.sync_copy(acc.at[pl.ds(base, last_stripe)],
                            out_hbm.at[c, pl.ds(base, last_stripe)])

    return sc_gat


def _head_matrix(a):
    heads, ch = a.shape
    return (jnp.eye(heads, dtype=a.dtype)[:, None, :]
            * a[:, :, None]).reshape(heads * ch, heads)


def kernel(x, edge_index, W1, a_src1, a_dst1, b1, W2, a_src2, a_dst2, b2):
    n = x.shape[0]
    e = edge_index.shape[1]
    heads, hid = a_src1.shape
    feat1 = heads * hid
    out_dim = W2.shape[1]
    dt1, dt2 = feat1 + 16, out_dim + 16

    workers = 32
    chunk1, chunk2 = 40, 80
    n_chunks1 = e // (workers * chunk1)
    n_chunks2 = e // (workers * chunk2)

    src_w1 = edge_index[0].reshape(workers, n_chunks1, chunk1)
    dst_w1 = edge_index[1].reshape(workers, n_chunks1, chunk1)
    src_w2 = edge_index[0].reshape(workers, n_chunks2, chunk2)
    dst_w2 = edge_index[1].reshape(workers, n_chunks2, chunk2)
    last_stripe = n - 15 * ((n // 16) & ~7)
    zeros1 = jnp.zeros((last_stripe, dt1), jnp.float32)
    zeros2 = jnp.zeros((last_stripe, dt2), jnp.float32)

    f32 = jnp.float32
    z8 = jnp.zeros((x.shape[1], 16 - heads), f32)
    wa1 = jnp.concatenate([W1, W1 @ _head_matrix(a_src1), z8], axis=1)
    wd1 = jnp.concatenate([W1 @ _head_matrix(a_dst1), z8], axis=1)

    t1, d1 = _proj(x, wa1, wd1)
    acc1 = _make_sc_gat(n, dt1, feat1, hid, n_chunks1, chunk1)(
        t1, d1, src_w1, dst_w1, zeros1)

    sel1 = jnp.concatenate(
        [jnp.zeros((feat1, feat1), f32),
         jnp.repeat(jnp.eye(heads, dtype=f32), hid, axis=1),
         jnp.zeros((16 - heads, feat1), f32)], axis=0)

    z15 = jnp.zeros((feat1, 15), f32)
    wa2 = jnp.concatenate([W2, W2 @ a_src2.T, z15], axis=1)
    wd2 = jnp.concatenate([W2 @ a_dst2.T, z15], axis=1)

    t2, d2 = _norm_proj(acc1[0], acc1[1], sel1, b1.reshape(1, feat1),
                        wa2, wd2)
    acc2 = _make_sc_gat(n, dt2, out_dim, out_dim, n_chunks2, chunk2)(
        t2, d2, src_w2, dst_w2, zeros2)

    sel2 = jnp.concatenate(
        [jnp.zeros((out_dim, out_dim), f32),
         jnp.ones((1, out_dim), f32),
         jnp.zeros((15, out_dim), f32)], axis=0)

    return _final(acc2[0], acc2[1], sel2, b2.reshape(1, out_dim))


# speedup vs baseline: 1.2350x; 1.0389x over previous
import functools

import jax
import jax.numpy as jnp
from jax import lax
from jax.experimental import pallas as pl
from jax.experimental.pallas import tpu as pltpu
from jax.experimental.pallas import tpu_sc as plsc

_EPS = 1e-16


def _proj_body(x_ref, wa_ref, wb_ref, oa_ref, ob_ref):
    x = x_ref[...]
    oa_ref[...] = jnp.dot(x, wa_ref[...], preferred_element_type=jnp.float32)
    ob_ref[...] = jnp.dot(x, wb_ref[...], preferred_element_type=jnp.float32)


def _proj(x, wa, wb, blk=2000):
    n, k = x.shape
    da, db = wa.shape[1], wb.shape[1]
    return pl.pallas_call(
        _proj_body,
        grid=(n // blk,),
        in_specs=[
            pl.BlockSpec((blk, k), lambda i: (i, 0)),
            pl.BlockSpec((k, da), lambda i: (0, 0)),
            pl.BlockSpec((k, db), lambda i: (0, 0)),
        ],
        out_specs=[
            pl.BlockSpec((blk, da), lambda i: (i, 0)),
            pl.BlockSpec((blk, db), lambda i: (i, 0)),
        ],
        out_shape=[
            jax.ShapeDtypeStruct((n, da), jnp.float32),
            jax.ShapeDtypeStruct((n, db), jnp.float32),
        ],
    )(x, wa, wb)


def _norm_proj_body(feat_w, a_ref, s_ref, b_ref, wa_ref, wb_ref,
                    oa_ref, ob_ref):
    hs = a_ref[0] + a_ref[1]
    den = jnp.dot(hs, s_ref[...], preferred_element_type=jnp.float32)
    h = jnp.maximum(hs[:, :feat_w] / (den + _EPS) + b_ref[...], 0.0)
    oa_ref[...] = jnp.dot(h, wa_ref[...], preferred_element_type=jnp.float32)
    ob_ref[...] = jnp.dot(h, wb_ref[...], preferred_element_type=jnp.float32)


def _norm_proj(a, sel, b, wa, wb, blk=2000):
    _, n, dt = a.shape
    feat_w = sel.shape[1]
    da, db = wa.shape[1], wb.shape[1]
    return pl.pallas_call(
        functools.partial(_norm_proj_body, feat_w),
        grid=(n // blk,),
        in_specs=[
            pl.BlockSpec((2, blk, dt), lambda i: (0, i, 0)),
            pl.BlockSpec((dt, feat_w), lambda i: (0, 0)),
            pl.BlockSpec((1, feat_w), lambda i: (0, 0)),
            pl.BlockSpec((feat_w, da), lambda i: (0, 0)),
            pl.BlockSpec((feat_w, db), lambda i: (0, 0)),
        ],
        out_specs=[
            pl.BlockSpec((blk, da), lambda i: (i, 0)),
            pl.BlockSpec((blk, db), lambda i: (i, 0)),
        ],
        out_shape=[
            jax.ShapeDtypeStruct((n, da), jnp.float32),
            jax.ShapeDtypeStruct((n, db), jnp.float32),
        ],
    )(a, sel, b, wa, wb)


def _final_body(feat_w, a_ref, s_ref, b_ref, o_ref):
    hs = a_ref[0] + a_ref[1]
    den = jnp.dot(hs, s_ref[...], preferred_element_type=jnp.float32)
    o_ref[...] = hs[:, :feat_w] / (den + _EPS) + b_ref[...]


def _final(a, sel, b, blk=2000):
    _, n, dt = a.shape
    feat_w = sel.shape[1]
    return pl.pallas_call(
        functools.partial(_final_body, feat_w),
        grid=(n // blk,),
        in_specs=[
            pl.BlockSpec((2, blk, dt), lambda i: (0, i, 0)),
            pl.BlockSpec((dt, feat_w), lambda i: (0, 0)),
            pl.BlockSpec((1, feat_w), lambda i: (0, 0)),
        ],
        out_specs=pl.BlockSpec((blk, feat_w), lambda i: (i, 0)),
        out_shape=jax.ShapeDtypeStruct((n, feat_w), jnp.float32),
    )(a, sel, b)


def _make_sc_gat(n_nodes, dt, feat_cols, head_w, n_chunks, chunk):
    n_heads = feat_cols // head_w
    vregs_per_head = head_w // 16
    epw = n_chunks * chunk
    mesh = plsc.VectorSubcoreMesh(core_axis_name="c", subcore_axis_name="s")
    stripe = (n_nodes // 16) & ~7
    last_stripe = n_nodes - 15 * stripe

    @functools.partial(
        pl.kernel,
        mesh=mesh,
        compiler_params=pltpu.CompilerParams(use_tc_tiling_on_sc=False),
        out_type=jax.ShapeDtypeStruct((2, n_nodes, dt), jnp.float32),
        scratch_types=[
            pltpu.VMEM((3, chunk), jnp.int32),
            pltpu.VMEM((3, chunk), jnp.int32),
            pltpu.VMEM((3, chunk, dt), jnp.float32),
            pltpu.VMEM((3, chunk, 16), jnp.float32),
            pltpu.VMEM((3, chunk, dt), jnp.float32),
            pltpu.VMEM_SHARED((n_nodes, dt), jnp.float32),
            pltpu.SemaphoreType.DMA,
            pltpu.SemaphoreType.DMA,
            pltpu.SemaphoreType.DMA,
            pltpu.SemaphoreType.DMA,
            pltpu.SemaphoreType.DMA,
            pltpu.SemaphoreType.DMA,
            pltpu.SemaphoreType.DMA,
            pltpu.SemaphoreType.DMA,
            pltpu.SemaphoreType.DMA,
            pltpu.SemaphoreType.DMA,
            pltpu.SemaphoreType.DMA,
            pltpu.SemaphoreType.DMA,
        ],
    )
    def sc_gat(t_hbm, d_hbm, src_hbm, dst_hbm, zeros_hbm, out_hbm,
               sidx, didx, trows, drows, srows, acc,
               gt0, gt1, gt2, gd0, gd1, gd2,
               ss0, ss1, ss2, is0, is1, is2):
        c = lax.axis_index("c")
        s = lax.axis_index("s")
        wid = s * 2 + c
        gts = (gt0, gt1, gt2)
        gds = (gd0, gd1, gd2)
        sss = (ss0, ss1, ss2)
        iss = (is0, is1, is2)

        def idx_stage(ch, b):
            off = pl.multiple_of(wid * epw + ch * chunk, 8)
            pltpu.async_copy(src_hbm.at[pl.ds(off, chunk)], sidx.at[b],
                             iss[b])
            pltpu.async_copy(dst_hbm.at[pl.ds(off, chunk)], didx.at[b],
                             iss[b])

        def idx_wait(b):
            pltpu.make_async_copy(src_hbm.at[pl.ds(0, chunk)], sidx.at[b],
                                  iss[b]).wait()
            pltpu.make_async_copy(dst_hbm.at[pl.ds(0, chunk)], didx.at[b],
                                  iss[b]).wait()

        def gather_start(ch, b):
            pltpu.async_copy(t_hbm.at[sidx.at[b]], trows.at[b], gts[b])
            pltpu.async_copy(d_hbm.at[didx.at[b]], drows.at[b], gds[b])

        def gather_wait(b):
            pltpu.make_async_copy(t_hbm.at[sidx.at[b]], trows.at[b],
                                  gts[b]).wait()
            pltpu.make_async_copy(d_hbm.at[didx.at[b]], drows.at[b],
                                  gds[b]).wait()

        def scatter_start(b):
            pltpu.async_copy(srows.at[b], acc.at[didx.at[b]], sss[b],
                             add=True)

        def scatter_wait(b):
            pltpu.make_async_copy(srows.at[b], acc.at[didx.at[b]],
                                  sss[b]).wait()

        idx_stage(0, 0)
        idx_stage(1, 1)

        base = s * stripe

        @pl.when(s < 15)
        def _():
            pltpu.sync_copy(zeros_hbm.at[pl.ds(0, stripe)],
                            acc.at[pl.ds(base, stripe)])

        @pl.when(s == 15)
        def _():
            pltpu.sync_copy(zeros_hbm, acc.at[pl.ds(base, last_stripe)])

        plsc.subcore_barrier()

        def compute(b):
            @plsc.parallel_loop(0, chunk, unroll=8)
            def edge_body(i):
                sv = trows[b, i, pl.ds(feat_cols, 16)]
                dv = drows[b, i, pl.ds(0, 16)]
                t = sv + dv
                w = jnp.exp(jnp.where(t >= 0, t, 0.2 * t))
                srows[b, i, pl.ds(feat_cols, 16)] = w
                for h in range(n_heads):
                    bj = w[h]
                    for v in range(vregs_per_head):
                        col = (h * vregs_per_head + v) * 16
                        srows[b, i, pl.ds(col, 16)] = (
                            trows[b, i, pl.ds(col, 16)] * bj)

        idx_wait(0)
        gather_start(0, 0)

        def sub_iter(ch, b, is_static):
            bn = (b + 1) % 3
            bp = (b + 2) % 3
            gather_wait(b)
            if not is_static or ch + 1 < n_chunks:
                idx_wait(bn)
                gather_start(ch + 1, bn)
            compute(b)
            if not is_static or ch > 0:
                scatter_wait(bp)
            scatter_start(b)
            if not is_static or ch + 2 < n_chunks:
                idx_stage(ch + 2, bp)

        for ch in range(0, 3):
            sub_iter(ch, ch % 3, True)

        def triple_body(t3, _):
            for b in (0, 1, 2):
                sub_iter(3 * t3 + b, b, False)
            return 0

        epi = 3 * (n_chunks // 3 - 1)
        lax.fori_loop(1, n_chunks // 3 - 1, triple_body, 0)
        for ch in range(epi, n_chunks):
            sub_iter(ch, ch % 3, True)
        scatter_wait((n_chunks - 1) % 3)
        plsc.subcore_barrier()

        @pl.when(s < 15)
        def _():
            pltpu.sync_copy(acc.at[pl.ds(base, stripe)],
                            out_hbm.at[c, pl.ds(base, stripe)])

        @pl.when(s == 15)
        def _():
            pltpu.sync_copy(acc.at[pl.ds(base, last_stripe)],
                            out_hbm.at[c, pl.ds(base, last_stripe)])

    return sc_gat


def _head_matrix(a):
    heads, ch = a.shape
    return (jnp.eye(heads, dtype=a.dtype)[:, None, :]
            * a[:, :, None]).reshape(heads * ch, heads)


def kernel(x, edge_index, W1, a_src1, a_dst1, b1, W2, a_src2, a_dst2, b2):
    n = x.shape[0]
    e = edge_index.shape[1]
    heads, hid = a_src1.shape
    feat1 = heads * hid
    out_dim = W2.shape[1]
    dt1, dt2 = feat1 + 16, out_dim + 16

    workers = 32
    chunk1, chunk2 = 40, 80
    n_chunks1 = e // (workers * chunk1)
    n_chunks2 = e // (workers * chunk2)

    src_e = edge_index[0]
    dst_e = edge_index[1]
    last_stripe = n - 15 * ((n // 16) & ~7)
    zeros1 = jnp.zeros((last_stripe, dt1), jnp.float32)
    zeros2 = jnp.zeros((last_stripe, dt2), jnp.float32)

    f32 = jnp.float32
    z8 = jnp.zeros((x.shape[1], 16 - heads), f32)
    wa1 = jnp.concatenate([W1, W1 @ _head_matrix(a_src1), z8], axis=1)
    wd1 = jnp.concatenate([W1 @ _head_matrix(a_dst1), z8], axis=1)

    t1, d1 = _proj(x, wa1, wd1)
    acc1 = _make_sc_gat(n, dt1, feat1, hid, n_chunks1, chunk1)(
        t1, d1, src_e, dst_e, zeros1)

    sel1 = jnp.concatenate(
        [jnp.zeros((feat1, feat1), f32),
         jnp.repeat(jnp.eye(heads, dtype=f32), hid, axis=1),
         jnp.zeros((16 - heads, feat1), f32)], axis=0)

    z15 = jnp.zeros((feat1, 15), f32)
    wa2 = jnp.concatenate([W2, W2 @ a_src2.T, z15], axis=1)
    wd2 = jnp.concatenate([W2 @ a_dst2.T, z15], axis=1)

    t2, d2 = _norm_proj(acc1, sel1, b1.reshape(1, feat1), wa2, wd2)
    acc2 = _make_sc_gat(n, dt2, out_dim, out_dim, n_chunks2, chunk2)(
        t2, d2, src_e, dst_e, zeros2)

    sel2 = jnp.concatenate(
        [jnp.zeros((out_dim, out_dim), f32),
         jnp.ones((1, out_dim), f32),
         jnp.zeros((15, out_dim), f32)], axis=0)

    return _final(acc2, sel2, b2.reshape(1, out_dim))


# speedup vs baseline: 1.2480x; 1.0106x over previous
import functools

import jax
import jax.numpy as jnp
import numpy as np
from jax import lax
from jax.experimental import pallas as pl
from jax.experimental.pallas import tpu as pltpu
from jax.experimental.pallas import tpu_sc as plsc

_EPS = 1e-16


def _interleave_perm(width):
    p = []
    for v in range(width // 32):
        for j in range(16):
            p += [32 * v + j, 32 * v + 16 + j]
    return np.array(p)


def _proj_body(x_ref, wf_ref, wa_ref, wd_ref, of_ref, oa_ref, od_ref):
    x = x_ref[...]
    of_ref[...] = jnp.dot(
        x, wf_ref[...], preferred_element_type=jnp.float32
    ).astype(jnp.bfloat16)
    oa_ref[...] = jnp.dot(x, wa_ref[...], preferred_element_type=jnp.float32)
    od_ref[...] = jnp.dot(x, wd_ref[...], preferred_element_type=jnp.float32)


def _proj(x, wf, wa, wd, blk=2000):
    n, k = x.shape
    df, da, dd = wf.shape[1], wa.shape[1], wd.shape[1]
    return pl.pallas_call(
        _proj_body,
        grid=(n // blk,),
        in_specs=[
            pl.BlockSpec((blk, k), lambda i: (i, 0)),
            pl.BlockSpec((k, df), lambda i: (0, 0)),
            pl.BlockSpec((k, da), lambda i: (0, 0)),
            pl.BlockSpec((k, dd), lambda i: (0, 0)),
        ],
        out_specs=[
            pl.BlockSpec((blk, df), lambda i: (i, 0)),
            pl.BlockSpec((blk, da), lambda i: (i, 0)),
            pl.BlockSpec((blk, dd), lambda i: (i, 0)),
        ],
        out_shape=[
            jax.ShapeDtypeStruct((n, df), jnp.bfloat16),
            jax.ShapeDtypeStruct((n, da), jnp.float32),
            jax.ShapeDtypeStruct((n, dd), jnp.float32),
        ],
    )(x, wf, wa, wd)


def _norm_proj_body(feat_w, a_ref, s_ref, b_ref, wf_ref, wa_ref, wd_ref,
                    of_ref, oa_ref, od_ref):
    hs = a_ref[0] + a_ref[1]
    den = jnp.dot(hs, s_ref[...], preferred_element_type=jnp.float32)
    h = jnp.maximum(hs[:, :feat_w] / (den + _EPS) + b_ref[...], 0.0)
    of_ref[...] = jnp.dot(
        h, wf_ref[...], preferred_element_type=jnp.float32
    ).astype(jnp.bfloat16)
    oa_ref[...] = jnp.dot(h, wa_ref[...], preferred_element_type=jnp.float32)
    od_ref[...] = jnp.dot(h, wd_ref[...], preferred_element_type=jnp.float32)


def _norm_proj(a, sel, b, wf, wa, wd, blk=2000):
    _, n, dt = a.shape
    feat_w = sel.shape[1]
    df, da, dd = wf.shape[1], wa.shape[1], wd.shape[1]
    return pl.pallas_call(
        functools.partial(_norm_proj_body, feat_w),
        grid=(n // blk,),
        in_specs=[
            pl.BlockSpec((2, blk, dt), lambda i: (0, i, 0)),
            pl.BlockSpec((dt, feat_w), lambda i: (0, 0)),
            pl.BlockSpec((1, feat_w), lambda i: (0, 0)),
            pl.BlockSpec((feat_w, df), lambda i: (0, 0)),
            pl.BlockSpec((feat_w, da), lambda i: (0, 0)),
            pl.BlockSpec((feat_w, dd), lambda i: (0, 0)),
        ],
        out_specs=[
            pl.BlockSpec((blk, df), lambda i: (i, 0)),
            pl.BlockSpec((blk, da), lambda i: (i, 0)),
            pl.BlockSpec((blk, dd), lambda i: (i, 0)),
        ],
        out_shape=[
            jax.ShapeDtypeStruct((n, df), jnp.bfloat16),
            jax.ShapeDtypeStruct((n, da), jnp.float32),
            jax.ShapeDtypeStruct((n, dd), jnp.float32),
        ],
    )(a, sel, b, wf, wa, wd)


def _final_body(feat_w, a_ref, s_ref, b_ref, o_ref):
    hs = a_ref[0] + a_ref[1]
    den = jnp.dot(hs, s_ref[...], preferred_element_type=jnp.float32)
    o_ref[...] = hs[:, :feat_w] / (den + _EPS) + b_ref[...]


def _final(a, sel, b, blk=2000):
    _, n, dt = a.shape
    feat_w = sel.shape[1]
    return pl.pallas_call(
        functools.partial(_final_body, feat_w),
        grid=(n // blk,),
        in_specs=[
            pl.BlockSpec((2, blk, dt), lambda i: (0, i, 0)),
            pl.BlockSpec((dt, feat_w), lambda i: (0, 0)),
            pl.BlockSpec((1, feat_w), lambda i: (0, 0)),
        ],
        out_specs=pl.BlockSpec((blk, feat_w), lambda i: (i, 0)),
        out_shape=jax.ShapeDtypeStruct((n, feat_w), jnp.float32),
    )(a, sel, b)


def _make_sc_gat(n_nodes, dt, feat_cols, head_w, n_chunks, chunk):
    epw = n_chunks * chunk
    mesh = plsc.VectorSubcoreMesh(core_axis_name="c", subcore_axis_name="s")
    stripe = (n_nodes // 16) & ~7
    last_stripe = n_nodes - 15 * stripe

    @functools.partial(
        pl.kernel,
        mesh=mesh,
        compiler_params=pltpu.CompilerParams(use_tc_tiling_on_sc=False),
        out_type=jax.ShapeDtypeStruct((2, n_nodes, dt), jnp.float32),
        scratch_types=[
            pltpu.VMEM((3, chunk), jnp.int32),
            pltpu.VMEM((3, chunk), jnp.int32),
            pltpu.VMEM((3, chunk, feat_cols // 2), jnp.int32),
            pltpu.VMEM((3, chunk, 16), jnp.float32),
            pltpu.VMEM((3, chunk, 16), jnp.float32),
            pltpu.VMEM((3, chunk, dt), jnp.float32),
            pltpu.VMEM_SHARED((n_nodes, dt), jnp.float32),
            pltpu.SemaphoreType.DMA,
            pltpu.SemaphoreType.DMA,
            pltpu.SemaphoreType.DMA,
            pltpu.SemaphoreType.DMA,
            pltpu.SemaphoreType.DMA,
            pltpu.SemaphoreType.DMA,
            pltpu.SemaphoreType.DMA,
            pltpu.SemaphoreType.DMA,
            pltpu.SemaphoreType.DMA,
            pltpu.SemaphoreType.DMA,
            pltpu.SemaphoreType.DMA,
            pltpu.SemaphoreType.DMA,
            pltpu.SemaphoreType.DMA,
            pltpu.SemaphoreType.DMA,
            pltpu.SemaphoreType.DMA,
        ],
    )
    def sc_gat(f_hbm, a_hbm, d_hbm, src_hbm, dst_hbm, zeros_hbm, out_hbm,
               sidx, didx, frows, arows, drows, srows, acc,
               gf0, gf1, gf2, ga0, ga1, ga2, gd0, gd1, gd2,
               ss0, ss1, ss2, is0, is1, is2):
        c = lax.axis_index("c")
        s = lax.axis_index("s")
        wid = s * 2 + c
        gfs = (gf0, gf1, gf2)
        gas = (ga0, ga1, ga2)
        gds = (gd0, gd1, gd2)
        sss = (ss0, ss1, ss2)
        iss = (is0, is1, is2)

        def idx_stage(ch, b):
            off = pl.multiple_of(wid * epw + ch * chunk, 8)
            pltpu.async_copy(src_hbm.at[pl.ds(off, chunk)], sidx.at[b],
                             iss[b])
            pltpu.async_copy(dst_hbm.at[pl.ds(off, chunk)], didx.at[b],
                             iss[b])

        def idx_wait(b):
            pltpu.make_async_copy(src_hbm.at[pl.ds(0, chunk)], sidx.at[b],
                                  iss[b]).wait()
            pltpu.make_async_copy(dst_hbm.at[pl.ds(0, chunk)], didx.at[b],
                                  iss[b]).wait()

        def gather_start(ch, b):
            pltpu.async_copy(f_hbm.at[sidx.at[b]], frows.at[b], gfs[b])
            pltpu.async_copy(a_hbm.at[sidx.at[b]], arows.at[b], gas[b])
            pltpu.async_copy(d_hbm.at[didx.at[b]], drows.at[b], gds[b])

        def gather_wait(b):
            pltpu.make_async_copy(f_hbm.at[sidx.at[b]], frows.at[b],
                                  gfs[b]).wait()
            pltpu.make_async_copy(a_hbm.at[sidx.at[b]], arows.at[b],
                                  gas[b]).wait()
            pltpu.make_async_copy(d_hbm.at[didx.at[b]], drows.at[b],
                                  gds[b]).wait()

        def scatter_start(b):
            pltpu.async_copy(srows.at[b], acc.at[didx.at[b]], sss[b],
                             add=True)

        def scatter_wait(b):
            pltpu.make_async_copy(srows.at[b], acc.at[didx.at[b]],
                                  sss[b]).wait()

        idx_stage(0, 0)
        idx_stage(1, 1)

        base = s * stripe

        @pl.when(s < 15)
        def _():
            pltpu.sync_copy(zeros_hbm.at[pl.ds(0, stripe)],
                            acc.at[pl.ds(base, stripe)])

        @pl.when(s == 15)
        def _():
            pltpu.sync_copy(zeros_hbm, acc.at[pl.ds(base, last_stripe)])

        plsc.subcore_barrier()

        def compute(b):
            @plsc.parallel_loop(0, chunk, unroll=8)
            def edge_body(i):
                sv = arows[b, i, pl.ds(0, 16)]
                dv = drows[b, i, pl.ds(0, 16)]
                t = sv + dv
                w = jnp.exp(jnp.where(t >= 0, t, 0.2 * t))
                srows[b, i, pl.ds(feat_cols, 16)] = w
                for v in range(feat_cols // 32):
                    u = frows[b, i, pl.ds(16 * v, 16)]
                    lo = lax.bitcast_convert_type(u << 16, jnp.float32)
                    hi = lax.bitcast_convert_type(u & jnp.int32(-65536),
                                                  jnp.float32)
                    srows[b, i, pl.ds(32 * v, 16)] = lo * w[32 * v // head_w]
                    srows[b, i, pl.ds(32 * v + 16, 16)] = (
                        hi * w[(32 * v + 16) // head_w])

        idx_wait(0)
        gather_start(0, 0)

        def sub_iter(ch, b, is_static):
            bn = (b + 1) % 3
            bp = (b + 2) % 3
            gather_wait(b)
            if not is_static or ch + 1 < n_chunks:
                idx_wait(bn)
                gather_start(ch + 1, bn)
            compute(b)
            if not is_static or ch > 0:
                scatter_wait(bp)
            scatter_start(b)
            if not is_static or ch + 2 < n_chunks:
                idx_stage(ch + 2, bp)

        for ch in range(0, 3):
            sub_iter(ch, ch % 3, True)

        def triple_body(t3, _):
            for b in (0, 1, 2):
                sub_iter(3 * t3 + b, b, False)
            return 0

        epi = 3 * (n_chunks // 3 - 1)
        lax.fori_loop(1, n_chunks // 3 - 1, triple_body, 0)
        for ch in range(epi, n_chunks):
            sub_iter(ch, ch % 3, True)
        scatter_wait((n_chunks - 1) % 3)
        plsc.subcore_barrier()

        @pl.when(s < 15)
        def _():
            pltpu.sync_copy(acc.at[pl.ds(base, stripe)],
                            out_hbm.at[c, pl.ds(base, stripe)])

        @pl.when(s == 15)
        def _():
            pltpu.sync_copy(acc.at[pl.ds(base, last_stripe)],
                            out_hbm.at[c, pl.ds(base, last_stripe)])

    return sc_gat


def _head_matrix(a):
    heads, ch = a.shape
    return (jnp.eye(heads, dtype=a.dtype)[:, None, :]
            * a[:, :, None]).reshape(heads * ch, heads)


def kernel(x, edge_index, W1, a_src1, a_dst1, b1, W2, a_src2, a_dst2, b2):
    n = x.shape[0]
    e = edge_index.shape[1]
    heads, hid = a_src1.shape
    feat1 = heads * hid
    out_dim = W2.shape[1]
    dt1, dt2 = feat1 + 16, out_dim + 16

    workers = 32
    chunk1, chunk2 = 40, 80
    n_chunks1 = e // (workers * chunk1)
    n_chunks2 = e // (workers * chunk2)

    src_e = edge_index[0]
    dst_e = edge_index[1]
    last_stripe = n - 15 * ((n // 16) & ~7)
    zeros1 = jnp.zeros((last_stripe, dt1), jnp.float32)
    zeros2 = jnp.zeros((last_stripe, dt2), jnp.float32)

    f32 = jnp.float32
    z8 = jnp.zeros((x.shape[1], 16 - heads), f32)
    wf1 = W1[:, _interleave_perm(feat1)]
    wa1 = jnp.concatenate([W1 @ _head_matrix(a_src1), z8], axis=1)
    wd1 = jnp.concatenate([W1 @ _head_matrix(a_dst1), z8], axis=1)

    t1, a1, d1 = _proj(x, wf1, wa1, wd1)
    t1i = lax.bitcast_convert_type(
        t1.reshape(n, feat1 // 2, 2), jnp.int32)
    acc1 = _make_sc_gat(n, dt1, feat1, hid, n_chunks1, chunk1)(
        t1i, a1, d1, src_e, dst_e, zeros1)

    sel1 = jnp.concatenate(
        [jnp.zeros((feat1, feat1), f32),
         jnp.repeat(jnp.eye(heads, dtype=f32), hid, axis=1),
         jnp.zeros((16 - heads, feat1), f32)], axis=0)

    z15 = jnp.zeros((feat1, 15), f32)
    wf2 = W2[:, _interleave_perm(out_dim)]
    wa2 = jnp.concatenate([W2 @ a_src2.T, z15], axis=1)
    wd2 = jnp.concatenate([W2 @ a_dst2.T, z15], axis=1)

    t2, a2, d2 = _norm_proj(acc1, sel1, b1.reshape(1, feat1), wf2, wa2, wd2)
    t2i = lax.bitcast_convert_type(
        t2.reshape(n, out_dim // 2, 2), jnp.int32)
    acc2 = _make_sc_gat(n, dt2, out_dim, out_dim, n_chunks2, chunk2)(
        t2i, a2, d2, src_e, dst_e, zeros2)

    sel2 = jnp.concatenate(
        [jnp.zeros((out_dim, out_dim), f32),
         jnp.ones((1, out_dim), f32),
         jnp.zeros((15, out_dim), f32)], axis=0)

    return _final(acc2, sel2, b2.reshape(1, out_dim))
